# baseline (speedup 1.0000x reference)
import jax
import jax.numpy as jnp
from jax.experimental import pallas as pl


def _argmax_block(x_ref, o_ref):
    x = x_ref[...]
    m = jnp.max(x, axis=1, keepdims=True)
    n = x.shape[1]
    ii = jax.lax.broadcasted_iota(jnp.int32, x.shape, 1)
    cand = jnp.where(x == m, ii, n)
    o_ref[0, 0, :] = jnp.min(cand, axis=1)


def kernel(input_0):
    b, s, n = input_0.shape
    rows = b * s
    block_rows = 256
    assert rows % block_rows == 0
    num_blocks = rows // block_rows
    x = input_0.reshape(rows, n)
    out = pl.pallas_call(
        _argmax_block,
        grid=(num_blocks,),
        in_specs=[pl.BlockSpec((block_rows, n), lambda i: (i, 0))],
        out_specs=pl.BlockSpec((1, 1, block_rows), lambda i: (i, 0, 0)),
        out_shape=jax.ShapeDtypeStruct((num_blocks, 1, block_rows), jnp.int32),
    )(x)
    return out.reshape(b, s).astype(jnp.int64)


# speedup vs baseline: 1.0849x; 1.0849x over previous
import jax
import jax.numpy as jnp
from jax.experimental import pallas as pl


def _argmax_block(x_ref, o_ref):
    x = x_ref[...]
    m = jnp.max(x, axis=1, keepdims=True)
    n = x.shape[1]
    ii = jax.lax.broadcasted_iota(jnp.int32, x.shape, 1)
    cand = jnp.where(x == m, ii, n)
    o_ref[0, 0, :] = jnp.min(cand, axis=1)


def kernel(input_0):
    b, s, n = input_0.shape
    rows = b * s
    block_rows = 512
    assert rows % block_rows == 0
    num_blocks = rows // block_rows
    x = input_0.reshape(rows, n)
    out = pl.pallas_call(
        _argmax_block,
        grid=(num_blocks,),
        in_specs=[pl.BlockSpec((block_rows, n), lambda i: (i, 0))],
        out_specs=pl.BlockSpec((1, 1, block_rows), lambda i: (i, 0, 0)),
        out_shape=jax.ShapeDtypeStruct((num_blocks, 1, block_rows), jnp.int32),
    )(x)
    return out.reshape(b, s).astype(jnp.int64)
